# Initial kernel scaffold; baseline (speedup 1.0000x reference)
import jax
import jax.numpy as jnp
from jax.experimental import pallas as pl


def kernel(features):
    raise NotImplementedError("write your pallas kernel here")


# speedup vs baseline: 2.1870x; 2.1870x over previous
import jax
import jax.numpy as jnp
from jax.experimental import pallas as pl
from jax.experimental.pallas import tpu as pltpu

_NUM_POINTS = 64
_FOLD = 8


def _fps_body(far_ref, pnc_ref, pt4_ref, sampled_ref, cent_ref):
    N = pnc_ref.shape[1]
    C = pnc_ref.shape[2]
    NL = N // _FOLD

    n_iota = (jax.lax.broadcasted_iota(jnp.int32, (_FOLD, NL), 0) * NL
              + jax.lax.broadcasted_iota(jnp.int32, (_FOLD, NL), 1))
    eye_mask = (jax.lax.broadcasted_iota(jnp.int32, (C, C), 0)
                == jax.lax.broadcasted_iota(jnp.int32, (C, C), 1))
    lane_np = jax.lax.broadcasted_iota(jnp.int32, (1, _NUM_POINTS), 1)

    pt4 = pt4_ref[0]

    def body(i, carry):
        f, distance, cent_vec = carry
        cent_vec = jnp.where(lane_np == i, f, cent_vec)
        row = pnc_ref[0, pl.ds(f, 1), :]
        sampled_ref[0, pl.ds(i, 1), :] = row
        col = jnp.sum(
            jnp.where(eye_mask, jnp.broadcast_to(row, (C, C)), 0.0),
            axis=1, keepdims=True)
        diff = pt4 - col
        dist = jnp.sum(diff * diff, axis=1)
        distance = jnp.where(dist < distance, dist, distance)
        m = jnp.max(distance)
        f_new = jnp.min(jnp.where(distance == m, n_iota, jnp.int32(N)))
        return f_new, distance, cent_vec

    f0 = far_ref[pl.program_id(0)]
    dist0 = jnp.full((_FOLD, NL), 1e10, jnp.float32)
    cent0 = jnp.zeros((1, _NUM_POINTS), jnp.int32)
    _, _, cent_vec = jax.lax.fori_loop(0, _NUM_POINTS, body, (f0, dist0, cent0))
    cent_ref[0] = cent_vec


def _fps_pallas(points, far0, interpret=False):
    B, N, C = points.shape
    NL = N // _FOLD
    pt4 = points.transpose(0, 2, 1).reshape(B, C, _FOLD, NL).transpose(0, 2, 1, 3)
    sampled, cent = pl.pallas_call(
        _fps_body,
        grid=(B,),
        in_specs=[
            pl.BlockSpec(memory_space=pltpu.SMEM),
            pl.BlockSpec((1, N, C), lambda b: (b, 0, 0)),
            pl.BlockSpec((1, _FOLD, C, NL), lambda b: (b, 0, 0, 0)),
        ],
        out_specs=[
            pl.BlockSpec((1, _NUM_POINTS, C), lambda b: (b, 0, 0)),
            pl.BlockSpec((1, 1, _NUM_POINTS), lambda b: (b, 0, 0)),
        ],
        out_shape=[
            jax.ShapeDtypeStruct((B, _NUM_POINTS, C), jnp.float32),
            jax.ShapeDtypeStruct((B, 1, _NUM_POINTS), jnp.int32),
        ],
        compiler_params=pltpu.CompilerParams(
            dimension_semantics=("arbitrary",)),
        interpret=interpret,
    )(far0, points, pt4)
    return sampled, cent.reshape(B, _NUM_POINTS)


@jax.jit
def kernel(features):
    B = features.shape[0]
    C = features.shape[-1]
    points = features.reshape(B, -1, C)
    N = points.shape[1]
    far0 = jax.random.randint(jax.random.key(1), (B,), 0, N, dtype=jnp.int32)
    return _fps_pallas(points, far0)
